# baseline (speedup 1.0000x reference)
import functools

import jax
import jax.numpy as jnp
from jax import lax
from jax.experimental import pallas as pl
from jax.experimental.pallas import tpu as pltpu
from jax.experimental.pallas import tpu_sc as plsc

EMB_DIM = 32
N_VOCAB = 1000
NUM_CORES = 2
NUM_SUBCORES = 16
NUM_WORKERS = NUM_CORES * NUM_SUBCORES
LANES = 16
BLK = 128
BLK_PAD = BLK + 1
ROW_PAD = EMB_DIM + 1


@functools.lru_cache(maxsize=None)
def _make_tiled_gather(batch: int, seq: int):
    assert batch == NUM_WORKERS * BLK
    mesh = plsc.VectorSubcoreMesh(core_axis_name="c", subcore_axis_name="s")
    e_hi = EMB_DIM // 8

    @functools.partial(
        pl.kernel,
        mesh=mesh,
        compiler_params=pltpu.CompilerParams(use_tc_tiling_on_sc=False,
                                             needs_layout_passes=False),
        out_type=jax.ShapeDtypeStruct((seq, e_hi, NUM_WORKERS, 8, BLK),
                                      jnp.float32),
        scratch_types=[
            pltpu.VMEM((N_VOCAB, ROW_PAD), jnp.float32),
            pltpu.VMEM((seq, BLK), jnp.int32),
            pltpu.VMEM((EMB_DIM, BLK_PAD), jnp.float32),
            pltpu.VMEM((EMB_DIM, BLK_PAD), jnp.float32),
            pltpu.SemaphoreType.DMA,
            pltpu.SemaphoreType.DMA,
        ],
    )
    def gather_kernel(sent_hbm, table_hbm, out_hbm, table_v, idx_v,
                      buf0, buf1, sem0, sem1):
        wid = lax.axis_index("s") * NUM_CORES + lax.axis_index("c")
        pltpu.sync_copy(table_hbm, table_v.at[:, pl.ds(0, EMB_DIM)])
        pltpu.sync_copy(sent_hbm.at[:, pl.ds(wid * BLK, BLK)], idx_v)

        bufs = (buf0, buf1)
        sems = (sem0, sem1)

        e_lo16 = lax.iota(jnp.int32, 16)
        consts = []
        for h in range(EMB_DIM // LANES):
            e_vec = e_lo16 + LANES * h
            consts.append(e_vec)

        def compute(s, buf):
            @plsc.parallel_loop(0, BLK, unroll=2)
            def _b_body(b):
                idx16 = idx_v[s, pl.ds((b // LANES) * LANES, LANES)]
                lane_j = jnp.full((LANES,), b % LANES, jnp.int32)
                idx_b = idx16.at[lane_j].get(mode="promise_in_bounds")
                bvec = jnp.full((LANES,), b, jnp.int32)
                for h in range(EMB_DIM // LANES):
                    val = plsc.load_gather(table_v, [idx_b, consts[h]])
                    plsc.store_scatter(buf, [consts[h], bvec], val)

        def start_store(s, b):
            h = pltpu.async_copy(
                bufs[b].at[pl.ds(0, 8), pl.ds(0, BLK)],
                out_hbm.at[s, 0, wid], sems[b])
            for g in range(1, e_hi):
                h = pltpu.async_copy(
                    bufs[b].at[pl.ds(8 * g, 8), pl.ds(0, BLK)],
                    out_hbm.at[s, g, wid], sems[b])
            return h

        def wait_store(s, b):
            for g in range(e_hi):
                pltpu.make_async_copy(
                    bufs[b].at[pl.ds(8 * g, 8), pl.ds(0, BLK)],
                    out_hbm.at[s, g, wid], sems[b]).wait()

        for s in (0, 1):
            compute(s, bufs[s])
            start_store(s, s)

        def loop_body(i, carry):
            s = 2 * i
            for b in range(2):
                wait_store(s + b - 2, b)
                compute(s + b, bufs[b])
                start_store(s + b, b)
            return carry

        lax.fori_loop(1, seq // 2, loop_body, 0)
        wait_store(seq - 2, 0)
        wait_store(seq - 1, 1)

    return gather_kernel


def kernel(sentence, table):
    batch, seq = sentence.shape
    sent_t = sentence.T.astype(jnp.int32)
    out5 = _make_tiled_gather(batch, seq)(sent_t, table)
    x = out5.transpose(2, 4, 0, 1, 3)
    return x.reshape(batch, seq, EMB_DIM)


# speedup vs baseline: 1.0646x; 1.0646x over previous
import functools

import jax
import jax.numpy as jnp
from jax import lax
from jax.experimental import pallas as pl
from jax.experimental.pallas import tpu as pltpu
from jax.experimental.pallas import tpu_sc as plsc

EMB_DIM = 32
N_VOCAB = 1000
NUM_CORES = 2
NUM_SUBCORES = 16
NUM_WORKERS = NUM_CORES * NUM_SUBCORES
LANES = 16
BLK = 128
BLK_PAD = BLK + 1
ROW_PAD = EMB_DIM + 1


@functools.lru_cache(maxsize=None)
def _make_tiled_gather(batch: int, seq: int):
    assert batch == NUM_WORKERS * BLK
    mesh = plsc.VectorSubcoreMesh(core_axis_name="c", subcore_axis_name="s")
    e_hi = EMB_DIM // 8

    @functools.partial(
        pl.kernel,
        mesh=mesh,
        compiler_params=pltpu.CompilerParams(use_tc_tiling_on_sc=False,
                                             needs_layout_passes=False),
        out_type=jax.ShapeDtypeStruct((seq, e_hi, NUM_WORKERS, 8, BLK),
                                      jnp.float32),
        scratch_types=[
            pltpu.VMEM((N_VOCAB, ROW_PAD), jnp.float32),
            pltpu.VMEM((seq, BLK), jnp.int32),
            pltpu.VMEM((EMB_DIM, BLK_PAD), jnp.float32),
            pltpu.VMEM((EMB_DIM, BLK_PAD), jnp.float32),
            pltpu.SemaphoreType.DMA,
            pltpu.SemaphoreType.DMA,
        ],
    )
    def gather_kernel(sent_hbm, table_hbm, out_hbm, table_v, idx_v,
                      buf0, buf1, sem0, sem1):
        wid = lax.axis_index("s") * NUM_CORES + lax.axis_index("c")
        pltpu.sync_copy(table_hbm, table_v.at[:, pl.ds(0, EMB_DIM)])
        pltpu.sync_copy(sent_hbm.at[:, pl.ds(wid * BLK, BLK)], idx_v)

        bufs = (buf0, buf1)
        sems = (sem0, sem1)

        e_lo16 = lax.iota(jnp.int32, 16)
        consts = []
        for h in range(EMB_DIM // LANES):
            e_vec = e_lo16 + LANES * h
            consts.append(e_vec)

        def compute(s, buf):
            @plsc.parallel_loop(0, BLK, unroll=4)
            def _b_body(b):
                idx16 = idx_v[s, pl.ds((b // LANES) * LANES, LANES)]
                lane_j = jnp.full((LANES,), b % LANES, jnp.int32)
                idx_b = idx16.at[lane_j].get(mode="promise_in_bounds")
                bvec = jnp.full((LANES,), b, jnp.int32)
                for h in range(EMB_DIM // LANES):
                    val = plsc.load_gather(table_v, [idx_b, consts[h]])
                    plsc.store_scatter(buf, [consts[h], bvec], val)

        def start_store(s, b):
            h = pltpu.async_copy(
                bufs[b].at[pl.ds(0, 8), pl.ds(0, BLK)],
                out_hbm.at[s, 0, wid], sems[b])
            for g in range(1, e_hi):
                h = pltpu.async_copy(
                    bufs[b].at[pl.ds(8 * g, 8), pl.ds(0, BLK)],
                    out_hbm.at[s, g, wid], sems[b])
            return h

        def wait_store(s, b):
            for g in range(e_hi):
                pltpu.make_async_copy(
                    bufs[b].at[pl.ds(8 * g, 8), pl.ds(0, BLK)],
                    out_hbm.at[s, g, wid], sems[b]).wait()

        for s in (0, 1):
            compute(s, bufs[s])
            start_store(s, s)

        def loop_body(i, carry):
            s = 2 * i
            for b in range(2):
                wait_store(s + b - 2, b)
                compute(s + b, bufs[b])
                start_store(s + b, b)
            return carry

        lax.fori_loop(1, seq // 2, loop_body, 0)
        wait_store(seq - 2, 0)
        wait_store(seq - 1, 1)

    return gather_kernel


def kernel(sentence, table):
    batch, seq = sentence.shape
    sent_t = sentence.T.astype(jnp.int32)
    out5 = _make_tiled_gather(batch, seq)(sent_t, table)
    x = out5.transpose(2, 4, 0, 1, 3)
    return x.reshape(batch, seq, EMB_DIM)
